# baseline (speedup 1.0000x reference)
import functools

import jax
import jax.numpy as jnp
from jax import lax
from jax.experimental import pallas as pl
from jax.experimental.pallas import tpu as pltpu
from jax.experimental.pallas import tpu_sc as plsc

Q = 1024
K_REAL = 100000
D = 128
Q_BLK = 256
K_TILE = 2048
N_TILES = 49
K_PAD = N_TILES * K_TILE
G = 16
NG = K_PAD // G
NV = NG // 16
NEG = -3.0e38
TOPK = 64

NW = 32
QH = Q // 8
QPW = QH // NW
CAPC = 1024
TRASH = CAPC + 8
GCH = 128
NCH_MAX = CAPC // GCH


def _phase1_body(q_ref, k_ref, s_ref, gm_ref):
    t = pl.program_id(1)
    s = lax.dot_general(q_ref[...], k_ref[...],
                        (((1,), (1,)), ((), ())),
                        preferred_element_type=jnp.float32)
    col = t * K_TILE + lax.broadcasted_iota(jnp.int32, (1, K_TILE), 1)
    s = jnp.where(col < K_REAL, s, NEG)
    s_ref[...] = s.reshape(Q_BLK, K_TILE // 128, 128)
    st = lax.dot_general(k_ref[...], q_ref[...],
                         (((1,), (1,)), ((), ())),
                         preferred_element_type=jnp.float32)
    colt = t * K_TILE + lax.broadcasted_iota(jnp.int32, (K_TILE, 1), 0)
    st = jnp.where(colt < K_REAL, st, NEG)
    gm_ref[...] = jnp.max(st.reshape(K_TILE // G, G, Q_BLK), axis=1)


def _phase1(queries, keys_padded):
    nq = queries.shape[0]
    return pl.pallas_call(
        _phase1_body,
        grid=(nq // Q_BLK, N_TILES),
        in_specs=[
            pl.BlockSpec((Q_BLK, D), lambda qb, t: (qb, 0)),
            pl.BlockSpec((K_TILE, D), lambda qb, t: (t, 0)),
        ],
        out_specs=[
            pl.BlockSpec((Q_BLK, K_TILE // 128, 128), lambda qb, t: (qb, t, 0)),
            pl.BlockSpec((K_TILE // G, Q_BLK), lambda qb, t: (t, qb)),
        ],
        out_shape=[
            jax.ShapeDtypeStruct((nq, K_PAD // 128, 128), jnp.float32),
            jax.ShapeDtypeStruct((NG, nq), jnp.float32),
        ],
    )(queries, keys_padded)


def _sc_select_body(gm_hbm, sc2_hbm, outs_hbm, outi_hbm,
                    gm_v, cand_v, gath_v, es_v, eid_v,
                    outs_b, outi_b, sem):
    i32 = jnp.int32
    IOTA = jnp.arange(16, dtype=i32)
    ZERO16 = jnp.zeros((16,), i32)
    ONE16 = jnp.full((16,), 1, i32)
    NEGV = jnp.full((16,), NEG, jnp.float32)
    wid = lax.axis_index("s") * 2 + lax.axis_index("c")

    def _gat(v, idx):
        return v.at[idx].get(mode="promise_in_bounds")

    def _rot(v, sh):
        return _gat(v, jnp.bitwise_and(IOTA + sh, 15))

    def _splat_max(v):
        for sh in (8, 4, 2, 1):
            v = jnp.maximum(v, _rot(v, sh))
        return v

    def _splat_min(v):
        for sh in (8, 4, 2, 1):
            v = jnp.minimum(v, _rot(v, sh))
        return v

    def per_query(ql, _):
        q = wid * QPW + ql
        pltpu.sync_copy(gm_hbm.at[q], gm_v)

        def blockmax(b):
            def mx(i, acc):
                return jnp.maximum(acc, gm_v[pl.ds((b * 98 + i) * 16, 16)])
            return lax.fori_loop(1, 98, mx, gm_v[pl.ds(b * 98 * 16, 16)], unroll=4)

        bm = jnp.minimum(jnp.minimum(blockmax(0), blockmax(1)),
                         jnp.minimum(blockmax(2), blockmax(3)))
        t_vec = _splat_min(bm)

        filler = jnp.full((16,), q * NG + (NG - 1), i32)

        def fill_c(i, c):
            cand_v[pl.ds(i * 16, 16)] = filler
            return c
        lax.fori_loop(0, (CAPC + 16) // 16, fill_c, 0)

        def fill_e(i, c):
            es_v[pl.ds(i * 16, 16)] = NEGV
            return c
        lax.fori_loop(0, (CAPC + 16) // 16, fill_e, 0)

        def comp(i, off):
            v = gm_v[pl.ds(i * 16, 16)]
            msk = v >= t_vec
            gid = q * NG + i * 16 + IOTA
            _, gid_s, _ = plsc.sort_key_val(IOTA, gid, mask=msk)
            cand_v[pl.ds(off, 16)] = gid_s
            cnt = plsc.all_reduce_population_count(msk)[0]
            return jnp.minimum(off + cnt, CAPC)

        ncand = lax.fori_loop(0, NV, comp, jnp.asarray(0, i32), unroll=2)

        nch = jnp.minimum((ncand + (GCH - 1)) // GCH, NCH_MAX)

        cp0 = pltpu.async_copy(sc2_hbm.at[cand_v.at[pl.ds(0, GCH)]],
                               gath_v.at[0], sem)

        def chunk(c, off2):
            pltpu.async_copy(
                sc2_hbm.at[cand_v.at[pl.ds(
                    jnp.minimum(c + 1, NCH_MAX - 1) * GCH, GCH)]],
                gath_v.at[(c + 1) % 2], sem)
            pltpu.make_async_copy(
                sc2_hbm.at[cand_v.at[pl.ds(0, GCH)]],
                gath_v.at[0], sem).wait()

            def row(r, o2):
                sv = gath_v[c % 2, r]
                msk = sv >= t_vec
                ra = c * GCH + r
                rb = (ra // 16) * 16
                crow = cand_v[pl.ds(rb, 16)]
                gidspl = _gat(crow, ONE16 * (ra - rb))
                eidv = (gidspl - q * NG) * 16 + IOTA
                _, es_s, _ = plsc.sort_key_val(IOTA, sv, mask=msk)
                _, eid_s, _ = plsc.sort_key_val(IOTA, eidv, mask=msk)
                es_v[pl.ds(o2, 16)] = es_s
                eid_v[pl.ds(o2, 16)] = eid_s
                cnt = plsc.all_reduce_population_count(msk)[0]
                return jnp.minimum(o2 + cnt, CAPC)

            return lax.fori_loop(0, GCH, row, off2)

        nelem = lax.fori_loop(0, nch, chunk, jnp.asarray(0, i32))
        pltpu.make_async_copy(
            sc2_hbm.at[cand_v.at[pl.ds(0, GCH)]], gath_v.at[0], sem).wait()
        nv2 = (nelem + 15) // 16

        def extract(j, _):
            def scan(r, carry):
                m, wr = carry
                v = es_v[pl.ds(r * 16, 16)]
                better = v > m
                wr = jnp.where(better, ONE16 * r, wr)
                m = jnp.maximum(m, v)
                return m, wr

            m, wrs = lax.fori_loop(0, nv2, scan, (NEGV, ZERO16))
            msv = _splat_max(m)
            pos = jnp.where(m == msv, wrs * 16 + IOTA,
                            jnp.full((16,), 1 << 30, i32))
            posm = _splat_min(pos)
            lane = posm % 16
            wrv = posm // 16
            erow = eid_v[pl.ds(wrv[0] * 16, 16)]
            wev = _gat(erow, lane)

            jb = (j // 16) * 16
            jmask = IOTA == (j - jb)
            ov = outs_b[pl.ds(jb, 16)]
            outs_b[pl.ds(jb, 16)] = jnp.where(jmask, msv, ov)
            oi = outi_b[pl.ds(jb, 16)]
            outi_b[pl.ds(jb, 16)] = jnp.where(jmask, wev, oi)

            rb = wrv[0] * 16
            rmask = IOTA == lane
            rv = es_v[pl.ds(rb, 16)]
            es_v[pl.ds(rb, 16)] = jnp.where(rmask, NEGV, rv)
            return _

        lax.fori_loop(0, TOPK, extract, 0)

        pltpu.sync_copy(outs_b, outs_hbm.at[q])
        pltpu.sync_copy(outi_b, outi_hbm.at[q])
        return _

    lax.fori_loop(0, QPW, per_query, 0)


@functools.partial(
    pl.kernel,
    out_type=[
        jax.ShapeDtypeStruct((QH, TOPK), jnp.float32),
        jax.ShapeDtypeStruct((QH, TOPK), jnp.int32),
    ],
    mesh=plsc.VectorSubcoreMesh(core_axis_name="c", subcore_axis_name="s"),
    compiler_params=pltpu.CompilerParams(needs_layout_passes=False, use_tc_tiling_on_sc=False),
    scratch_types=[
        pltpu.VMEM((NG,), jnp.float32),
        pltpu.VMEM((CAPC + 32,), jnp.int32),
        pltpu.VMEM((2, GCH, 16), jnp.float32),
        pltpu.VMEM((CAPC + 32,), jnp.float32),
        pltpu.VMEM((CAPC + 32,), jnp.int32),
        pltpu.VMEM((TOPK,), jnp.float32),
        pltpu.VMEM((TOPK,), jnp.int32),
        pltpu.SemaphoreType.DMA,
    ],
)
def _sc_select(gm_hbm, sc2_hbm, outs_hbm, outi_hbm, *scratch):
    _sc_select_body(gm_hbm, sc2_hbm, outs_hbm, outi_hbm, *scratch)


@jax.jit
def _run(queries, keys):
    keys_padded = jnp.pad(keys, ((0, K_PAD - K_REAL), (0, 0)))
    outs = []
    for h in range(8):
        scores, gm_t = _phase1(queries[h * QH:(h + 1) * QH], keys_padded)
        sc2 = scores.reshape(QH * NG, G)
        outs.append(_sc_select(gm_t.T, sc2))
    return (jnp.concatenate([o[0] for o in outs]),
            jnp.concatenate([o[1] for o in outs]))


def kernel(queries, keys, to_rerank, k):
    top_scores, top_idx = _run(queries, keys)
    return top_scores, top_idx


# speedup vs baseline: 1.0817x; 1.0817x over previous
import functools

import jax
import jax.numpy as jnp
from jax import lax
from jax.experimental import pallas as pl
from jax.experimental.pallas import tpu as pltpu
from jax.experimental.pallas import tpu_sc as plsc

Q = 1024
K_REAL = 100000
D = 128
Q_BLK = 256
K_TILE = 2048
N_TILES = 49
K_PAD = N_TILES * K_TILE
G = 16
NG = K_PAD // G
NV = NG // 16
NEG = -3.0e38
TOPK = 64

NW = 32
QH = Q // 4
QPW = QH // NW
CAPC = 1024
TRASH = CAPC + 8
GCH = 128
NCH_MAX = CAPC // GCH


def _phase1_body(q_ref, k_ref, s_ref, gm_ref):
    t = pl.program_id(1)
    s = lax.dot_general(q_ref[...], k_ref[...],
                        (((1,), (1,)), ((), ())),
                        preferred_element_type=jnp.float32)
    col = t * K_TILE + lax.broadcasted_iota(jnp.int32, (1, K_TILE), 1)
    s = jnp.where(col < K_REAL, s, NEG)
    s_ref[...] = s.reshape(Q_BLK, K_TILE // 128, 128)
    st = lax.dot_general(k_ref[...], q_ref[...],
                         (((1,), (1,)), ((), ())),
                         preferred_element_type=jnp.float32)
    colt = t * K_TILE + lax.broadcasted_iota(jnp.int32, (K_TILE, 1), 0)
    st = jnp.where(colt < K_REAL, st, NEG)
    gm_ref[...] = jnp.max(st.reshape(K_TILE // G, G, Q_BLK), axis=1)


def _phase1(queries, keys_padded):
    nq = queries.shape[0]
    return pl.pallas_call(
        _phase1_body,
        grid=(nq // Q_BLK, N_TILES),
        in_specs=[
            pl.BlockSpec((Q_BLK, D), lambda qb, t: (qb, 0)),
            pl.BlockSpec((K_TILE, D), lambda qb, t: (t, 0)),
        ],
        out_specs=[
            pl.BlockSpec((Q_BLK, K_TILE // 128, 128), lambda qb, t: (qb, t, 0)),
            pl.BlockSpec((K_TILE // G, Q_BLK), lambda qb, t: (t, qb)),
        ],
        out_shape=[
            jax.ShapeDtypeStruct((nq, K_PAD // 128, 128), jnp.float32),
            jax.ShapeDtypeStruct((NG, nq), jnp.float32),
        ],
    )(queries, keys_padded)


def _sc_select_body(gm_hbm, sc2_hbm, outs_hbm, outi_hbm,
                    gm_v, cand_v, gath_v, es_v, eid_v,
                    outs_b, outi_b, sem):
    i32 = jnp.int32
    IOTA = jnp.arange(16, dtype=i32)
    ZERO16 = jnp.zeros((16,), i32)
    ONE16 = jnp.full((16,), 1, i32)
    NEGV = jnp.full((16,), NEG, jnp.float32)
    wid = lax.axis_index("s") * 2 + lax.axis_index("c")

    def _gat(v, idx):
        return v.at[idx].get(mode="promise_in_bounds")

    def _rot(v, sh):
        return _gat(v, jnp.bitwise_and(IOTA + sh, 15))

    def _splat_max(v):
        for sh in (8, 4, 2, 1):
            v = jnp.maximum(v, _rot(v, sh))
        return v

    def _splat_min(v):
        for sh in (8, 4, 2, 1):
            v = jnp.minimum(v, _rot(v, sh))
        return v

    def per_query(ql, _):
        q = wid * QPW + ql
        pltpu.sync_copy(gm_hbm.at[q], gm_v)

        def blockmax(b):
            def mx(i, acc):
                return jnp.maximum(acc, gm_v[pl.ds((b * 98 + i) * 16, 16)])
            return lax.fori_loop(1, 98, mx, gm_v[pl.ds(b * 98 * 16, 16)], unroll=4)

        bm = jnp.minimum(jnp.minimum(blockmax(0), blockmax(1)),
                         jnp.minimum(blockmax(2), blockmax(3)))
        t_vec = _splat_min(bm)

        filler = jnp.full((16,), q * NG + (NG - 1), i32)

        def fill_c(i, c):
            cand_v[pl.ds(i * 16, 16)] = filler
            return c
        lax.fori_loop(0, (CAPC + 16) // 16, fill_c, 0)

        def fill_e(i, c):
            es_v[pl.ds(i * 16, 16)] = NEGV
            return c
        lax.fori_loop(0, (CAPC + 16) // 16, fill_e, 0)

        def comp(i, off):
            v = gm_v[pl.ds(i * 16, 16)]
            msk = v >= t_vec
            gid = q * NG + i * 16 + IOTA
            _, gid_s, _ = plsc.sort_key_val(IOTA, gid, mask=msk)
            cand_v[pl.ds(off, 16)] = gid_s
            cnt = plsc.all_reduce_population_count(msk)[0]
            return jnp.minimum(off + cnt, CAPC)

        ncand = lax.fori_loop(0, NV, comp, jnp.asarray(0, i32), unroll=2)

        nch = jnp.minimum((ncand + (GCH - 1)) // GCH, NCH_MAX)

        cp0 = pltpu.async_copy(sc2_hbm.at[cand_v.at[pl.ds(0, GCH)]],
                               gath_v.at[0], sem)

        def chunk(c, off2):
            pltpu.async_copy(
                sc2_hbm.at[cand_v.at[pl.ds(
                    jnp.minimum(c + 1, NCH_MAX - 1) * GCH, GCH)]],
                gath_v.at[(c + 1) % 2], sem)
            pltpu.make_async_copy(
                sc2_hbm.at[cand_v.at[pl.ds(0, GCH)]],
                gath_v.at[0], sem).wait()

            def row(r, o2):
                sv = gath_v[c % 2, r]
                msk = sv >= t_vec
                ra = c * GCH + r
                rb = (ra // 16) * 16
                crow = cand_v[pl.ds(rb, 16)]
                gidspl = _gat(crow, ONE16 * (ra - rb))
                eidv = (gidspl - q * NG) * 16 + IOTA
                _, es_s, _ = plsc.sort_key_val(IOTA, sv, mask=msk)
                _, eid_s, _ = plsc.sort_key_val(IOTA, eidv, mask=msk)
                es_v[pl.ds(o2, 16)] = es_s
                eid_v[pl.ds(o2, 16)] = eid_s
                cnt = plsc.all_reduce_population_count(msk)[0]
                return jnp.minimum(o2 + cnt, CAPC)

            return lax.fori_loop(0, GCH, row, off2)

        nelem = lax.fori_loop(0, nch, chunk, jnp.asarray(0, i32))
        pltpu.make_async_copy(
            sc2_hbm.at[cand_v.at[pl.ds(0, GCH)]], gath_v.at[0], sem).wait()
        nv2 = (nelem + 15) // 16

        def extract(j, _):
            def scan(r, carry):
                m, wr = carry
                v = es_v[pl.ds(r * 16, 16)]
                better = v > m
                wr = jnp.where(better, ONE16 * r, wr)
                m = jnp.maximum(m, v)
                return m, wr

            m, wrs = lax.fori_loop(0, nv2, scan, (NEGV, ZERO16))
            msv = _splat_max(m)
            pos = jnp.where(m == msv, wrs * 16 + IOTA,
                            jnp.full((16,), 1 << 30, i32))
            posm = _splat_min(pos)
            lane = posm % 16
            wrv = posm // 16
            erow = eid_v[pl.ds(wrv[0] * 16, 16)]
            wev = _gat(erow, lane)

            jb = (j // 16) * 16
            jmask = IOTA == (j - jb)
            ov = outs_b[pl.ds(jb, 16)]
            outs_b[pl.ds(jb, 16)] = jnp.where(jmask, msv, ov)
            oi = outi_b[pl.ds(jb, 16)]
            outi_b[pl.ds(jb, 16)] = jnp.where(jmask, wev, oi)

            rb = wrv[0] * 16
            rmask = IOTA == lane
            rv = es_v[pl.ds(rb, 16)]
            es_v[pl.ds(rb, 16)] = jnp.where(rmask, NEGV, rv)
            return _

        lax.fori_loop(0, TOPK, extract, 0)

        pltpu.sync_copy(outs_b, outs_hbm.at[q])
        pltpu.sync_copy(outi_b, outi_hbm.at[q])
        return _

    lax.fori_loop(0, QPW, per_query, 0)


@functools.partial(
    pl.kernel,
    out_type=[
        jax.ShapeDtypeStruct((QH, TOPK), jnp.float32),
        jax.ShapeDtypeStruct((QH, TOPK), jnp.int32),
    ],
    mesh=plsc.VectorSubcoreMesh(core_axis_name="c", subcore_axis_name="s"),
    compiler_params=pltpu.CompilerParams(needs_layout_passes=False, use_tc_tiling_on_sc=False),
    scratch_types=[
        pltpu.VMEM((NG,), jnp.float32),
        pltpu.VMEM((CAPC + 32,), jnp.int32),
        pltpu.VMEM((2, GCH, 16), jnp.float32),
        pltpu.VMEM((CAPC + 32,), jnp.float32),
        pltpu.VMEM((CAPC + 32,), jnp.int32),
        pltpu.VMEM((TOPK,), jnp.float32),
        pltpu.VMEM((TOPK,), jnp.int32),
        pltpu.SemaphoreType.DMA,
    ],
)
def _sc_select(gm_hbm, sc2_hbm, outs_hbm, outi_hbm, *scratch):
    _sc_select_body(gm_hbm, sc2_hbm, outs_hbm, outi_hbm, *scratch)


@jax.jit
def _run(queries, keys):
    keys_padded = jnp.pad(keys, ((0, K_PAD - K_REAL), (0, 0)))
    outs = []
    for h in range(4):
        scores, gm_t = _phase1(queries[h * QH:(h + 1) * QH], keys_padded)
        sc2 = scores.reshape(QH * NG, G)
        outs.append(_sc_select(gm_t.T, sc2))
    return (jnp.concatenate([o[0] for o in outs]),
            jnp.concatenate([o[1] for o in outs]))


def kernel(queries, keys, to_rerank, k):
    top_scores, top_idx = _run(queries, keys)
    return top_scores, top_idx
